# baseline (speedup 1.0000x reference)
import jax
import jax.numpy as jnp
from jax import lax
from jax.experimental import pallas as pl
from jax.experimental.pallas import tpu as pltpu
from jax.experimental.pallas import tpu_sc as plsc

EMB = 16
BATCH = 16384
NW = 32
BW = BATCH // NW
GRP = 1
LOGG = GRP.bit_length() - 1
LPG = 16 // GRP
NPH = 16
NGRP = BW // GRP
SLABW = 128
SHIFT = SLABW.bit_length() - 1
REG = 1e-05


def _take(v, ix):
    return v.at[ix].get(mode="promise_in_bounds")


def _body(user_hbm, item_hbm, users_hbm, pos_hbm, neg_hbm, out_hbm,
          idx_u, idx_p, idx_n, flat_u, flat_p, flat_n,
          slab_u, slab_p, slab_n, vout, *sems):
    c = lax.axis_index("c")
    s = lax.axis_index("s")
    wid = s * 2 + c

    row0 = pl.multiple_of((wid >> 1) << 3, 8)
    pltpu.sync_copy(users_hbm.at[pl.ds(row0, 8)], idx_u)
    pltpu.sync_copy(pos_hbm.at[pl.ds(row0, 8)], idx_p)
    pltpu.sync_copy(neg_hbm.at[pl.ds(row0, 8)], idx_n)

    lane = lax.iota(jnp.int32, 16)
    elo = lane & (GRP - 1)
    lane_mask = lane < GRP
    lb4 = (wid & 1) == 1

    for raw, flat in ((idx_u, flat_u), (idx_p, flat_p), (idx_n, flat_n)):
        for r in range(4):
            for k in range(8):
                va = raw[r, pl.ds(k * 16, 16)]
                vb = raw[r + 4, pl.ds(k * 16, 16)]
                flat[0, pl.ds(r * 128 + k * 16, 16)] = jnp.where(lb4, vb, va)

    def group_vecs(g):
        jb = (g // LPG) << 4
        rot = (lane + ((g % LPG) << LOGG)) & 15
        vu = _take(flat_u[0, pl.ds(jb, 16)], rot)
        vp = _take(flat_p[0, pl.ds(jb, 16)], rot)
        vn = _take(flat_n[0, pl.ds(jb, 16)], rot)
        return vu, vp, vn

    def fire_group(g, b):
        vu, vp, vn = group_vecs(g)
        bu = (vu >> SHIFT) << SHIFT
        bp = (vp >> SHIFT) << SHIFT
        bn = (vn >> SHIFT) << SHIFT
        sem = sems[b]
        for e in range(GRP):
            cu = pl.multiple_of(bu[e], 128)
            cp = pl.multiple_of(bp[e], 128)
            cn = pl.multiple_of(bn[e], 128)
            pltpu.async_copy(user_hbm.at[:, pl.ds(cu, SLABW)], slab_u.at[b, e], sem)
            pltpu.async_copy(item_hbm.at[:, pl.ds(cp, SLABW)], slab_p.at[b, e], sem)
            pltpu.async_copy(item_hbm.at[:, pl.ds(cn, SLABW)], slab_n.at[b, e], sem)

    def drain_group(b):
        sem = sems[b]
        for e in range(GRP):
            pltpu.make_async_copy(
                user_hbm.at[:, pl.ds(0, SLABW)], slab_u.at[b, e], sem).wait()
            pltpu.make_async_copy(
                item_hbm.at[:, pl.ds(0, SLABW)], slab_p.at[b, e], sem).wait()
            pltpu.make_async_copy(
                item_hbm.at[:, pl.ds(0, SLABW)], slab_n.at[b, e], sem).wait()

    dbase = (lane >> LOGG) << LOGG

    def extract_group(g, b, carry):
        mf, sq = carry
        vu, vp, vn = group_vecs(g)
        co_u = _take(vu & (SLABW - 1), elo)
        co_p = _take(vp & (SLABW - 1), elo)
        co_n = _take(vn & (SLABW - 1), elo)
        x = jnp.zeros((16,), jnp.float32)
        sqg = jnp.zeros((16,), jnp.float32)
        for d in range(EMB * GRP // 16):
            dv = dbase + d
            u = plsc.load_gather(slab_u.at[b], [elo, dv, co_u])
            p = plsc.load_gather(slab_p.at[b], [elo, dv, co_p])
            n = plsc.load_gather(slab_n.at[b], [elo, dv, co_n])
            x = x + u * (p - n)
            sqg = sqg + (u * u + (p * p + n * n))
        fold = 8
        while fold >= GRP:
            x = x + _take(x, lane ^ fold)
            sqg = sqg + _take(sqg, lane ^ fold)
            fold //= 2
        y = jnp.exp(-jnp.abs(x))
        t = y / (y + 2.0)
        t2 = t * t
        poly = 1.0 + t2 * (1.0 / 3.0 + t2 * (1.0 / 5.0 + t2 * (1.0 / 7.0 + t2 * (1.0 / 9.0 + t2 * (1.0 / 11.0)))))
        sp = jnp.maximum(-x, 0.0) + 2.0 * t * poly
        mf = mf + jnp.where(lane_mask, sp, 0.0)
        sq = sq + jnp.where(lane_mask, sqg, 0.0)
        return (mf, sq)

    zero = jnp.zeros((16,), jnp.float32)
    for b in range(NPH - 1):
        fire_group(b, b)

    def loop_body(h, carry):
        for b in range(NPH):
            g = h * NPH + b
            fire_group(g + NPH - 1, (b + NPH - 1) % NPH)
            drain_group(b)
            carry = extract_group(g, b, carry)
        return carry

    mf_acc, sq_acc = lax.fori_loop(0, NGRP // NPH - 1, loop_body, (zero, zero))
    carry = (mf_acc, sq_acc)
    for b in range(NPH):
        g = NGRP - NPH + b
        if b == 0:
            fire_group(NGRP - 1, (NGRP - 1) % NPH)
        drain_group(b)
        carry = extract_group(g, b, carry)
    mf_acc, sq_acc = carry

    def _allsum(v):
        for h in (8, 4, 2, 1):
            v = v + _take(v, lane ^ h)
        return v

    vec = jnp.where(lane == 0, _allsum(mf_acc),
                    jnp.where(lane == 1, _allsum(sq_acc), 0.0))
    zvec = jnp.zeros((16,), jnp.float32)
    vout[0, :] = vec
    for r in range(1, 8):
        vout[r, :] = zvec
    pltpu.sync_copy(vout, out_hbm.at[wid])


def kernel(user_emb, item_emb, users, pos_items, neg_items):
    mesh = plsc.VectorSubcoreMesh(core_axis_name="c", subcore_axis_name="s")
    part = pl.kernel(
        _body,
        mesh=mesh,
        compiler_params=pltpu.CompilerParams(
            use_tc_tiling_on_sc=True, needs_layout_passes=False),
        out_type=jax.ShapeDtypeStruct((NW, 8, EMB), jnp.float32),
        scratch_types=[
            pltpu.VMEM((8, 128), jnp.int32),
            pltpu.VMEM((8, 128), jnp.int32),
            pltpu.VMEM((8, 128), jnp.int32),
            pltpu.VMEM((1, BW), jnp.int32),
            pltpu.VMEM((1, BW), jnp.int32),
            pltpu.VMEM((1, BW), jnp.int32),
            pltpu.VMEM((NPH, GRP, EMB, SLABW), jnp.float32),
            pltpu.VMEM((NPH, GRP, EMB, SLABW), jnp.float32),
            pltpu.VMEM((NPH, GRP, EMB, SLABW), jnp.float32),
            pltpu.VMEM((8, EMB), jnp.float32),
        ] + [pltpu.SemaphoreType.DMA] * NPH,
    )(
        user_emb.T,
        item_emb.T,
        users.astype(jnp.int32).reshape(128, 128),
        pos_items.astype(jnp.int32).reshape(128, 128),
        neg_items.astype(jnp.int32).reshape(128, 128),
    )
    mf_loss = jnp.sum(part[:, 0, 0]) / BATCH
    emb_loss = REG * jnp.sum(part[:, 0, 1])
    return (mf_loss, emb_loss)


# speedup vs baseline: 1.0291x; 1.0291x over previous
import jax
import jax.numpy as jnp
from jax import lax
from jax.experimental import pallas as pl
from jax.experimental.pallas import tpu as pltpu
from jax.experimental.pallas import tpu_sc as plsc

EMB = 16
BATCH = 16384
NW = 32
BW = BATCH // NW
GRP = 2
LOGG = GRP.bit_length() - 1
LPG = 16 // GRP
NPH = 8
NGRP = BW // GRP
SLABW = 128
SHIFT = SLABW.bit_length() - 1
REG = 1e-05


def _take(v, ix):
    return v.at[ix].get(mode="promise_in_bounds")


def _body(user_hbm, item_hbm, users_hbm, pos_hbm, neg_hbm, out_hbm,
          idx_u, idx_p, idx_n, flat_u, flat_p, flat_n,
          slab_u, slab_p, slab_n, vout, *sems):
    c = lax.axis_index("c")
    s = lax.axis_index("s")
    wid = s * 2 + c

    row0 = pl.multiple_of((wid >> 1) << 3, 8)
    pltpu.sync_copy(users_hbm.at[pl.ds(row0, 8)], idx_u)
    pltpu.sync_copy(pos_hbm.at[pl.ds(row0, 8)], idx_p)
    pltpu.sync_copy(neg_hbm.at[pl.ds(row0, 8)], idx_n)

    lane = lax.iota(jnp.int32, 16)
    elo = lane & (GRP - 1)
    lane_mask = lane < GRP
    lb4 = (wid & 1) == 1

    for raw, flat in ((idx_u, flat_u), (idx_p, flat_p), (idx_n, flat_n)):
        for r in range(4):
            for k in range(8):
                va = raw[r, pl.ds(k * 16, 16)]
                vb = raw[r + 4, pl.ds(k * 16, 16)]
                flat[0, pl.ds(r * 128 + k * 16, 16)] = jnp.where(lb4, vb, va)

    def group_vecs(g):
        jb = (g // LPG) << 4
        rot = (lane + ((g % LPG) << LOGG)) & 15
        vu = _take(flat_u[0, pl.ds(jb, 16)], rot)
        vp = _take(flat_p[0, pl.ds(jb, 16)], rot)
        vn = _take(flat_n[0, pl.ds(jb, 16)], rot)
        return vu, vp, vn

    def fire_group(g, b):
        vu, vp, vn = group_vecs(g)
        bu = (vu >> SHIFT) << SHIFT
        bp = (vp >> SHIFT) << SHIFT
        bn = (vn >> SHIFT) << SHIFT
        sem = sems[b]
        for e in range(GRP):
            cu = pl.multiple_of(bu[e], 128)
            cp = pl.multiple_of(bp[e], 128)
            cn = pl.multiple_of(bn[e], 128)
            pltpu.async_copy(user_hbm.at[:, pl.ds(cu, SLABW)], slab_u.at[b, e], sem)
            pltpu.async_copy(item_hbm.at[:, pl.ds(cp, SLABW)], slab_p.at[b, e], sem)
            pltpu.async_copy(item_hbm.at[:, pl.ds(cn, SLABW)], slab_n.at[b, e], sem)

    def drain_group(b):
        sem = sems[b]
        for e in range(GRP):
            pltpu.make_async_copy(
                user_hbm.at[:, pl.ds(0, SLABW)], slab_u.at[b, e], sem).wait()
            pltpu.make_async_copy(
                item_hbm.at[:, pl.ds(0, SLABW)], slab_p.at[b, e], sem).wait()
            pltpu.make_async_copy(
                item_hbm.at[:, pl.ds(0, SLABW)], slab_n.at[b, e], sem).wait()

    dbase = (lane >> LOGG) << LOGG

    def extract_group(g, b, carry):
        mf, sq = carry
        vu, vp, vn = group_vecs(g)
        co_u = _take(vu & (SLABW - 1), elo)
        co_p = _take(vp & (SLABW - 1), elo)
        co_n = _take(vn & (SLABW - 1), elo)
        x = jnp.zeros((16,), jnp.float32)
        sqg = jnp.zeros((16,), jnp.float32)
        for d in range(EMB * GRP // 16):
            dv = dbase + d
            u = plsc.load_gather(slab_u.at[b], [elo, dv, co_u])
            p = plsc.load_gather(slab_p.at[b], [elo, dv, co_p])
            n = plsc.load_gather(slab_n.at[b], [elo, dv, co_n])
            x = x + u * (p - n)
            sqg = sqg + (u * u + (p * p + n * n))
        fold = 8
        while fold >= GRP:
            x = x + _take(x, lane ^ fold)
            sqg = sqg + _take(sqg, lane ^ fold)
            fold //= 2
        y = jnp.exp(-jnp.abs(x))
        t = y / (y + 2.0)
        t2 = t * t
        poly = 1.0 + t2 * (1.0 / 3.0 + t2 * (1.0 / 5.0 + t2 * (1.0 / 7.0 + t2 * (1.0 / 9.0 + t2 * (1.0 / 11.0)))))
        sp = jnp.maximum(-x, 0.0) + 2.0 * t * poly
        mf = mf + jnp.where(lane_mask, sp, 0.0)
        sq = sq + jnp.where(lane_mask, sqg, 0.0)
        return (mf, sq)

    zero = jnp.zeros((16,), jnp.float32)
    for b in range(NPH - 1):
        fire_group(b, b)

    def loop_body(h, carry):
        for b in range(NPH):
            g = h * NPH + b
            fire_group(g + NPH - 1, (b + NPH - 1) % NPH)
            drain_group(b)
            carry = extract_group(g, b, carry)
        return carry

    mf_acc, sq_acc = lax.fori_loop(0, NGRP // NPH - 1, loop_body, (zero, zero))
    carry = (mf_acc, sq_acc)
    for b in range(NPH):
        g = NGRP - NPH + b
        if b == 0:
            fire_group(NGRP - 1, (NGRP - 1) % NPH)
        drain_group(b)
        carry = extract_group(g, b, carry)
    mf_acc, sq_acc = carry

    def _allsum(v):
        for h in (8, 4, 2, 1):
            v = v + _take(v, lane ^ h)
        return v

    vec = jnp.where(lane == 0, _allsum(mf_acc),
                    jnp.where(lane == 1, _allsum(sq_acc), 0.0))
    zvec = jnp.zeros((16,), jnp.float32)
    vout[0, :] = vec
    for r in range(1, 8):
        vout[r, :] = zvec
    pltpu.sync_copy(vout, out_hbm.at[wid])


def kernel(user_emb, item_emb, users, pos_items, neg_items):
    mesh = plsc.VectorSubcoreMesh(core_axis_name="c", subcore_axis_name="s")
    part = pl.kernel(
        _body,
        mesh=mesh,
        compiler_params=pltpu.CompilerParams(
            use_tc_tiling_on_sc=True, needs_layout_passes=False),
        out_type=jax.ShapeDtypeStruct((NW, 8, EMB), jnp.float32),
        scratch_types=[
            pltpu.VMEM((8, 128), jnp.int32),
            pltpu.VMEM((8, 128), jnp.int32),
            pltpu.VMEM((8, 128), jnp.int32),
            pltpu.VMEM((1, BW), jnp.int32),
            pltpu.VMEM((1, BW), jnp.int32),
            pltpu.VMEM((1, BW), jnp.int32),
            pltpu.VMEM((NPH, GRP, EMB, SLABW), jnp.float32),
            pltpu.VMEM((NPH, GRP, EMB, SLABW), jnp.float32),
            pltpu.VMEM((NPH, GRP, EMB, SLABW), jnp.float32),
            pltpu.VMEM((8, EMB), jnp.float32),
        ] + [pltpu.SemaphoreType.DMA] * NPH,
    )(
        user_emb.T,
        item_emb.T,
        users.astype(jnp.int32).reshape(128, 128),
        pos_items.astype(jnp.int32).reshape(128, 128),
        neg_items.astype(jnp.int32).reshape(128, 128),
    )
    mf_loss = jnp.sum(part[:, 0, 0]) / BATCH
    emb_loss = REG * jnp.sum(part[:, 0, 1])
    return (mf_loss, emb_loss)
